# Initial kernel scaffold; baseline (speedup 1.0000x reference)
import jax
import jax.numpy as jnp
from jax.experimental import pallas as pl


def kernel(acts, group_labels):
    raise NotImplementedError("write your pallas kernel here")


# speedup vs baseline: 10.7506x; 10.7506x over previous
import functools

import jax
import jax.numpy as jnp
from jax import lax
from jax.experimental import pallas as pl
from jax.experimental.pallas import tpu as pltpu
from jax.experimental.pallas import tpu_sc as plsc

N = 500000
NBINS = 64
L = 16
NW = 32
CH = 15616
TAIL_BASE = NW * CH
TAIL_VECS = (N - TAIL_BASE) // L
ROWS = 144
TABW = ROWS * L


def _sc_histogram(acts, labels):
  mesh = plsc.VectorSubcoreMesh(core_axis_name="c", subcore_axis_name="s")

  @functools.partial(
      pl.kernel,
      mesh=mesh,
      out_type=jax.ShapeDtypeStruct((NW, TABW), jnp.float32),
      compiler_params=pltpu.CompilerParams(needs_layout_passes=False),
      scratch_types=[
          pltpu.VMEM((CH,), jnp.float32),
          pltpu.VMEM((CH,), jnp.int32),
          pltpu.VMEM((L,), jnp.float32),
          pltpu.VMEM((L,), jnp.int32),
          pltpu.VMEM((TABW,), jnp.float32),
          pltpu.SemaphoreType.DMA,
          pltpu.SemaphoreType.DMA,
          pltpu.SemaphoreType.DMA,
      ],
  )
  def k(acts_hbm, lbl_hbm, out_hbm, acts_v, lbl_v, acts_t, lbl_t, tab, sem0, sem1, sem2):
    cid = lax.axis_index("c")
    sid = lax.axis_index("s")
    wid = cid * 16 + sid
    off = wid * CH

    H = CH // 2
    cpa0 = pltpu.async_copy(acts_hbm.at[pl.ds(off, H)], acts_v.at[pl.ds(0, H)], sem0)
    cpl0 = pltpu.async_copy(lbl_hbm.at[pl.ds(off, H)], lbl_v.at[pl.ds(0, H)], sem0)
    cpa1 = pltpu.async_copy(acts_hbm.at[pl.ds(off + H, H)], acts_v.at[pl.ds(H, H)], sem1)
    cpl1 = pltpu.async_copy(lbl_hbm.at[pl.ds(off + H, H)], lbl_v.at[pl.ds(H, H)], sem1)

    is_tail = wid < TAIL_VECS
    toff = TAIL_BASE + wid * L

    @pl.when(is_tail)
    def _():
      pltpu.async_copy(acts_hbm.at[pl.ds(toff, L)], acts_t, sem2)
      pltpu.async_copy(lbl_hbm.at[pl.ds(toff, L)], lbl_t, sem2)

    zeros = jnp.zeros((L,), jnp.float32)

    def zbody(i, carry):
      tab[pl.ds(i * L, L)] = zeros
      return carry

    lax.fori_loop(0, ROWS, zbody, 0)

    lane144 = lax.iota(jnp.int32, L) * ROWS

    def prep(x, g):
      s = 1.0 / (1.0 + jnp.exp(-x))
      t = jnp.maximum(s * 64.0 - 64e-6, 0.0)
      bi = t.astype(jnp.int32)
      frac = t - bi.astype(jnp.float32)
      base = lane144 + g * 72 + bi
      return base, frac

    def accum(x, g):
      base, frac = prep(x, g)
      plsc.addupdate_scatter(tab, [base], 1.0 - frac)
      plsc.addupdate_scatter(tab, [base + 1], frac)

    K = 2

    def body(i):
      o = i * (L * K)
      pairs = [prep(acts_v[pl.ds(o + k * L, L)], lbl_v[pl.ds(o + k * L, L)])
               for k in range(K)]
      for base, frac in pairs:
        plsc.addupdate_scatter(tab, [base], 1.0 - frac)
        plsc.addupdate_scatter(tab, [base + 1], frac)

    NIT = CH // (L * K)
    cpa0.wait()
    cpl0.wait()
    plsc.parallel_loop(0, NIT // 2)(body)
    cpa1.wait()
    cpl1.wait()
    plsc.parallel_loop(NIT // 2, NIT)(body)

    @pl.when(is_tail)
    def _():
      pltpu.make_async_copy(acts_hbm.at[pl.ds(toff, L)], acts_t, sem2).wait()
      pltpu.make_async_copy(lbl_hbm.at[pl.ds(toff, L)], lbl_t, sem2).wait()
      accum(acts_t[...], lbl_t[...])

    pltpu.sync_copy(tab, out_hbm.at[wid])

  return k(acts, labels)


def _emd_body(p_ref, out_ref):
  p = p_ref[...]
  colsum = jnp.sum(p, axis=0, keepdims=True)
  rows = colsum[:, 0:ROWS]
  for l in range(1, L):
    rows = rows + colsum[:, l * ROWS:(l + 1) * ROWS]
  a = rows[:, 0:NBINS]
  b = rows[:, 72:72 + NBINS]
  d = a / jnp.sum(a) - b / jnp.sum(b)
  kk = lax.broadcasted_iota(jnp.int32, (NBINS, NBINS), 0)
  jj = lax.broadcasted_iota(jnp.int32, (NBINS, NBINS), 1)
  mask = (jj <= kk).astype(jnp.float32)
  c = jnp.sum(jnp.broadcast_to(d, (NBINS, NBINS)) * mask, axis=1, keepdims=True)
  m = (lax.broadcasted_iota(jnp.int32, (NBINS, 1), 0) < NBINS - 1).astype(jnp.float32)
  out_ref[...] = jnp.sum(jnp.abs(c) * m, axis=0, keepdims=True)


def kernel(acts, group_labels):
  partials = _sc_histogram(acts, group_labels)
  loss = pl.pallas_call(
      _emd_body,
      out_shape=jax.ShapeDtypeStruct((1, 1), jnp.float32),
  )(partials)
  return loss[0, 0]
